# baseline (speedup 1.0000x reference)
import numpy as np
import jax
import jax.numpy as jnp
from jax.experimental import pallas as pl
from jax.experimental.pallas import tpu as pltpu

_B = 64
_P = 128
_N = 8128
_KNN = 16
_LAYERS = 3
_RB = 16


def _build_layout():
    offs = (np.arange(128) * (np.arange(128) - 1)) // 2
    idx = np.zeros((_B, _P), np.int32)
    sid = np.full((_B, _P), -1, np.int32)
    qp1 = np.zeros((_B, _P), np.float32)
    inv = np.zeros((_N,), np.int32)
    for b in range(_B):
        slot = 0
        for s in (b, 127 - b):
            if s == 0:
                continue
            idx[b, slot:slot + s] = offs[s] + np.arange(s)
            sid[b, slot:slot + s] = s
            qp1[b, slot:slot + s] = min(_KNN, s - 2) + 1
            inv[offs[s]:offs[s] + s] = b * _P + slot + np.arange(s)
            slot += s
    return idx, sid, qp1, inv


_IDX, _SID, _QP1, _INV = _build_layout()


def _silu(v):
    h = 0.5 * v
    return h + h * jnp.tanh(h)


def _tpT(a):
    e = jnp.eye(128, dtype=jnp.float32)
    return jax.lax.dot_general(a, e, (((0,), (0,)), ((), ())),
                               preferred_element_type=jnp.float32)


def _egnn_kernel(z_ref, x_ref, sidc_ref, sidr_ref, qp1_ref, emb_ref,
                 W1a, W1b, w1d, b1, W2, b2,
                 pW1, pb1, pW2c, pb2,
                 nW1a, nW1b, nb1, nW2, nb2,
                 out_ref,
                 H_s, Xt_s, D_s, T_s, Af_s, cnt_s, Xs_s, Ms_s,
                 Di_s, Ha_s):
    zrow = z_ref[0]
    cls = jax.lax.broadcasted_iota(jnp.int32, (128, 128), 0)
    oneT = (cls == zrow).astype(jnp.float32)
    H_s[...] = jax.lax.dot_general(oneT, emb_ref[...],
                                   (((0,), (0,)), ((), ())),
                                   preferred_element_type=jnp.float32)
    Xt_s[...] = x_ref[0]

    sidc = sidc_ref[0]
    sidr = sidr_ref[0]
    ii = jax.lax.broadcasted_iota(jnp.int32, (128, 128), 0)
    jj = jax.lax.broadcasted_iota(jnp.int32, (128, 128), 1)
    vmask = (sidc == sidr) & (sidc >= 0) & (ii != jj)

    def compute_D():
        Xt = Xt_s[...]
        XtT = _tpT(Xt)
        acc = None
        for c in range(3):
            d = Xt[:, c:c + 1] - XtT[c:c + 1, :]
            sq = d * d
            acc = sq if acc is None else acc + sq
        D_s[...] = jnp.sqrt(acc)

    compute_D()
    Di_s[...] = jnp.where(vmask, D_s[...], jnp.inf)

    def tblk(b):
        r0 = b * 8
        Dr = Di_s[pl.ds(r0, 8), :]
        qr = qp1_ref[0, pl.ds(r0, 8), :]
        cmp = (Dr[:, None, :] <= Dr[:, :, None]).astype(jnp.float32)
        c = jnp.sum(cmp, axis=2)
        Tr = jnp.min(jnp.where(c >= qr, Dr, jnp.inf), axis=1, keepdims=True)
        T_s[pl.ds(r0, 8), :] = Tr

    for b in range(16):
        tblk(b)

    M = (D_s[...] < T_s[...]) & vmask
    Mf = jnp.where(M, 1.0, 0.0)
    Af = jnp.minimum(Mf + _tpT(Mf), 1.0)
    Af_s[...] = Af
    cnt_s[...] = jnp.maximum(jnp.sum(Af, axis=1, keepdims=True), 1.0)

    for l in range(_LAYERS):
        if l > 0:
            compute_D()
        H = H_s[...]
        Ha_s[...] = (jnp.dot(H, W1a[l], preferred_element_type=jnp.float32)
                     + b1[l])
        Hb = jnp.dot(H, W1b[l], preferred_element_type=jnp.float32)
        Xs_s[...] = jnp.zeros((128, 3), jnp.float32)
        Ms_s[...] = jnp.zeros((128, 128), jnp.float32)
        Xt = Xt_s[...]
        w1dl = w1d[l][None]
        W2l = W2[l]
        b2l = b2[l]
        pW1l = pW1[l]
        pb1l = pb1[l]
        pW2cl = pW2c[l]
        pb2l = pb2[l][0, 0]

        def blk(bi):
            r0 = bi * _RB
            Dr = D_s[pl.ds(r0, _RB), :]
            Har = Ha_s[pl.ds(r0, _RB), :]
            P = _silu(Har[:, None, :] + Hb[None, :, :]
                      + Dr[:, :, None] * w1dl)
            P2 = P.reshape(_RB * 128, 128)
            m = _silu(jnp.dot(P2, W2l, preferred_element_type=jnp.float32)
                      + b2l)
            t = _silu(jnp.dot(m, pW1l, preferred_element_type=jnp.float32)
                      + pb1l)
            w = jnp.dot(t, pW2cl, preferred_element_type=jnp.float32) + pb2l
            wre = w.reshape(_RB, 128)
            Ar = Af_s[pl.ds(r0, _RB), :]
            AW = Ar * wre
            xr = Xt_s[pl.ds(r0, _RB), :]
            xs = (jnp.dot(AW, Xt, preferred_element_type=jnp.float32)
                  - jnp.sum(AW, axis=1, keepdims=True) * xr)
            Xs_s[pl.ds(r0, _RB), :] = xs
            m3 = m.reshape(_RB, 128, 128)
            Ms_s[pl.ds(r0, _RB), :] = jnp.sum(m3 * Ar[:, :, None], axis=1)

        for bi in range(128 // _RB):
            blk(bi)

        cnt = cnt_s[...]
        Mi = Ms_s[...] / cnt
        Hn = H + (jnp.dot(
            _silu(jnp.dot(H, nW1a[l], preferred_element_type=jnp.float32)
                  + jnp.dot(Mi, nW1b[l], preferred_element_type=jnp.float32)
                  + nb1[l]),
            nW2[l], preferred_element_type=jnp.float32) + nb2[l])
        H_s[...] = Hn
        Xt_s[...] = Xt + Xs_s[...] / cnt

    out_ref[0] = Xt_s[...]


def _full(shape):
    nd = len(shape)
    return pl.BlockSpec(shape, lambda i, _nd=nd: (0,) * _nd)


def kernel(x, z, num_atoms, edges, emb, params):
    idx = jnp.asarray(_IDX)
    inv = jnp.asarray(_INV)
    xp = x[idx]
    zp = z[idx].astype(jnp.int32).reshape(_B, 1, _P)
    sidc = jnp.asarray(_SID).reshape(_B, _P, 1)
    sidr = jnp.asarray(_SID).reshape(_B, 1, _P)
    qp1 = jnp.asarray(_QP1).reshape(_B, _P, 1)
    emb_p = jnp.pad(emb, ((0, 128 - emb.shape[0]), (0, 0)))

    st = lambda k: jnp.stack([p[k] for p in params])
    W1 = st("eW1")
    W1a, W1b, w1d = W1[:, 0:128], W1[:, 128:256], W1[:, 256:257]
    b1 = st("eb1")[:, None, :]
    W2 = st("eW2")
    b2 = st("eb2")[:, None, :]
    pW1 = st("pW1")
    pb1 = st("pb1")[:, None, :]
    pW2c = st("pW2")
    pb2 = jnp.stack([jnp.broadcast_to(p["pb2"][None, :], (1, 128))
                     for p in params])
    nW1 = st("nW1")
    nW1a, nW1b = nW1[:, 0:128], nW1[:, 128:256]
    nb1 = st("nb1")[:, None, :]
    nW2 = st("nW2")
    nb2 = st("nb2")[:, None, :]

    out_p = pl.pallas_call(
        _egnn_kernel,
        grid=(_B,),
        in_specs=[
            pl.BlockSpec((1, 1, _P), lambda i: (i, 0, 0)),
            pl.BlockSpec((1, _P, 3), lambda i: (i, 0, 0)),
            pl.BlockSpec((1, _P, 1), lambda i: (i, 0, 0)),
            pl.BlockSpec((1, 1, _P), lambda i: (i, 0, 0)),
            pl.BlockSpec((1, _P, 1), lambda i: (i, 0, 0)),
            _full((128, 128)),
            _full(W1a.shape), _full(W1b.shape), _full(w1d.shape),
            _full(b1.shape), _full(W2.shape), _full(b2.shape),
            _full(pW1.shape), _full(pb1.shape), _full(pW2c.shape),
            _full(pb2.shape),
            _full(nW1a.shape), _full(nW1b.shape), _full(nb1.shape),
            _full(nW2.shape), _full(nb2.shape),
        ],
        out_specs=pl.BlockSpec((1, _P, 3), lambda i: (i, 0, 0)),
        out_shape=jax.ShapeDtypeStruct((_B, _P, 3), jnp.float32),
        scratch_shapes=[
            pltpu.VMEM((128, 128), jnp.float32),
            pltpu.VMEM((128, 3), jnp.float32),
            pltpu.VMEM((128, 128), jnp.float32),
            pltpu.VMEM((128, 1), jnp.float32),
            pltpu.VMEM((128, 128), jnp.float32),
            pltpu.VMEM((128, 1), jnp.float32),
            pltpu.VMEM((128, 3), jnp.float32),
            pltpu.VMEM((128, 128), jnp.float32),
            pltpu.VMEM((128, 128), jnp.float32),
            pltpu.VMEM((128, 128), jnp.float32),
        ],
        compiler_params=pltpu.CompilerParams(
            dimension_semantics=("arbitrary",)),
    )(zp, xp, sidc, sidr, qp1, emb_p, W1a, W1b, w1d, b1, W2, b2,
      pW1, pb1, pW2c, pb2, nW1a, nW1b, nb1, nW2, nb2)

    return out_p.reshape(_B * _P, 3)[inv]


# speedup vs baseline: 1.3362x; 1.3362x over previous
import numpy as np
import jax
import jax.numpy as jnp
from jax.experimental import pallas as pl
from jax.experimental.pallas import tpu as pltpu

_B = 64
_P = 128
_N = 8128
_KNN = 16
_LAYERS = 3
_RB = 16


def _build_layout():
    offs = (np.arange(128) * (np.arange(128) - 1)) // 2
    idx = np.zeros((_B, _P), np.int32)
    sid = np.full((_B, _P), -1, np.int32)
    qp1 = np.zeros((_B, _P), np.float32)
    inv = np.zeros((_N,), np.int32)
    for b in range(_B):
        slot = 0
        for s in (b, 127 - b):
            if s == 0:
                continue
            idx[b, slot:slot + s] = offs[s] + np.arange(s)
            sid[b, slot:slot + s] = s
            qp1[b, slot:slot + s] = min(_KNN, s - 2) + 1
            inv[offs[s]:offs[s] + s] = b * _P + slot + np.arange(s)
            slot += s
    return idx, sid, qp1, inv


_IDX, _SID, _QP1, _INV = _build_layout()


def _silu(v):
    h = 0.5 * v
    return h + h * jnp.tanh(h)


def _tpT(a):
    e = jnp.eye(128, dtype=jnp.float32)
    return jax.lax.dot_general(a, e, (((0,), (0,)), ((), ())),
                               preferred_element_type=jnp.float32)


def _egnn_kernel(z_ref, x_ref, sidc_ref, sidr_ref, qp1_ref, emb_ref,
                 W1a, W1b, w1d, b1, W2, b2,
                 pW1, pb1, pW2c, pb2,
                 nW1a, nW1b, nb1, nW2, nb2,
                 out_ref,
                 H_s, Xt_s, D_s, T_s, Af_s, cnt_s, Xs_s, Ms_s,
                 Di_s, Ha_s):
    zrow = z_ref[0]
    cls = jax.lax.broadcasted_iota(jnp.int32, (128, 128), 0)
    oneT = (cls == zrow).astype(jnp.float32)
    H_s[...] = jax.lax.dot_general(oneT, emb_ref[...],
                                   (((0,), (0,)), ((), ())),
                                   preferred_element_type=jnp.float32)
    Xt_s[...] = x_ref[0]

    sidc = sidc_ref[0]
    sidr = sidr_ref[0]
    ii = jax.lax.broadcasted_iota(jnp.int32, (128, 128), 0)
    jj = jax.lax.broadcasted_iota(jnp.int32, (128, 128), 1)
    vmask = (sidc == sidr) & (sidc >= 0) & (ii != jj)

    def compute_D():
        Xt = Xt_s[...]
        XtT = _tpT(Xt)
        acc = None
        for c in range(3):
            d = Xt[:, c:c + 1] - XtT[c:c + 1, :]
            sq = d * d
            acc = sq if acc is None else acc + sq
        D_s[...] = jnp.sqrt(acc)

    compute_D()
    Di_s[...] = jnp.where(vmask, D_s[...], jnp.inf)

    def tblk(b):
        r0 = b * 8
        Dr = Di_s[pl.ds(r0, 8), :]
        qr = qp1_ref[0, pl.ds(r0, 8), :]
        cmp = (Dr[:, None, :] <= Dr[:, :, None]).astype(jnp.float32)
        c = jnp.sum(cmp, axis=2)
        Tr = jnp.min(jnp.where(c >= qr, Dr, jnp.inf), axis=1, keepdims=True)
        T_s[pl.ds(r0, 8), :] = Tr

    def tblk2(b, carry):
        tblk(b * 2)
        tblk(b * 2 + 1)
        return carry

    jax.lax.fori_loop(0, 8, tblk2, 0)

    M = (D_s[...] < T_s[...]) & vmask
    Mf = jnp.where(M, 1.0, 0.0)
    Af = jnp.minimum(Mf + _tpT(Mf), 1.0)
    Af_s[...] = Af
    cnt_s[...] = jnp.maximum(jnp.sum(Af, axis=1, keepdims=True), 1.0)

    for l in range(_LAYERS):
        if l > 0:
            compute_D()
        H = H_s[...]
        Ha_s[...] = (jnp.dot(H, W1a[l], preferred_element_type=jnp.float32)
                     + b1[l])
        Hb = jnp.dot(H, W1b[l], preferred_element_type=jnp.float32)
        Xs_s[...] = jnp.zeros((128, 3), jnp.float32)
        Ms_s[...] = jnp.zeros((128, 128), jnp.float32)
        Xt = Xt_s[...]
        w1dl = w1d[l][None]
        W2l = W2[l]
        b2l = b2[l]
        pW1l = pW1[l]
        pb1l = pb1[l]
        pW2cl = pW2c[l]
        pb2l = pb2[l][0, 0]

        def blk(bi):
            r0 = bi * _RB
            Dr = D_s[pl.ds(r0, _RB), :]
            Har = Ha_s[pl.ds(r0, _RB), :]
            P = _silu(Har[:, None, :] + Hb[None, :, :]
                      + Dr[:, :, None] * w1dl)
            P2 = P.reshape(_RB * 128, 128)
            m = _silu(jnp.dot(P2, W2l, preferred_element_type=jnp.float32)
                      + b2l)
            t = _silu(jnp.dot(m, pW1l, preferred_element_type=jnp.float32)
                      + pb1l)
            w = jnp.dot(t, pW2cl, preferred_element_type=jnp.float32) + pb2l
            wre = w.reshape(_RB, 128)
            Ar = Af_s[pl.ds(r0, _RB), :]
            AW = Ar * wre
            xr = Xt_s[pl.ds(r0, _RB), :]
            xs = (jnp.dot(AW, Xt, preferred_element_type=jnp.float32)
                  - jnp.sum(AW, axis=1, keepdims=True) * xr)
            Xs_s[pl.ds(r0, _RB), :] = xs
            m3 = m.reshape(_RB, 128, 128)
            Ms_s[pl.ds(r0, _RB), :] = jnp.sum(m3 * Ar[:, :, None], axis=1)

        def blk2(bi, carry):
            blk(bi * 2)
            blk(bi * 2 + 1)
            return carry

        jax.lax.fori_loop(0, 64 // _RB, blk2, 0)

        cnt = cnt_s[...]
        Mi = Ms_s[...] / cnt
        Hn = H + (jnp.dot(
            _silu(jnp.dot(H, nW1a[l], preferred_element_type=jnp.float32)
                  + jnp.dot(Mi, nW1b[l], preferred_element_type=jnp.float32)
                  + nb1[l]),
            nW2[l], preferred_element_type=jnp.float32) + nb2[l])
        H_s[...] = Hn
        Xt_s[...] = Xt + Xs_s[...] / cnt

    out_ref[0] = Xt_s[...]


def _full(shape):
    nd = len(shape)
    return pl.BlockSpec(shape, lambda i, _nd=nd: (0,) * _nd)


def kernel(x, z, num_atoms, edges, emb, params):
    idx = jnp.asarray(_IDX)
    inv = jnp.asarray(_INV)
    xp = x[idx]
    zp = z[idx].astype(jnp.int32).reshape(_B, 1, _P)
    sidc = jnp.asarray(_SID).reshape(_B, _P, 1)
    sidr = jnp.asarray(_SID).reshape(_B, 1, _P)
    qp1 = jnp.asarray(_QP1).reshape(_B, _P, 1)
    emb_p = jnp.pad(emb, ((0, 128 - emb.shape[0]), (0, 0)))

    st = lambda k: jnp.stack([p[k] for p in params])
    W1 = st("eW1")
    W1a, W1b, w1d = W1[:, 0:128], W1[:, 128:256], W1[:, 256:257]
    b1 = st("eb1")[:, None, :]
    W2 = st("eW2")
    b2 = st("eb2")[:, None, :]
    pW1 = st("pW1")
    pb1 = st("pb1")[:, None, :]
    pW2c = st("pW2")
    pb2 = jnp.stack([jnp.broadcast_to(p["pb2"][None, :], (1, 128))
                     for p in params])
    nW1 = st("nW1")
    nW1a, nW1b = nW1[:, 0:128], nW1[:, 128:256]
    nb1 = st("nb1")[:, None, :]
    nW2 = st("nW2")
    nb2 = st("nb2")[:, None, :]

    out_p = pl.pallas_call(
        _egnn_kernel,
        grid=(_B,),
        in_specs=[
            pl.BlockSpec((1, 1, _P), lambda i: (i, 0, 0)),
            pl.BlockSpec((1, _P, 3), lambda i: (i, 0, 0)),
            pl.BlockSpec((1, _P, 1), lambda i: (i, 0, 0)),
            pl.BlockSpec((1, 1, _P), lambda i: (i, 0, 0)),
            pl.BlockSpec((1, _P, 1), lambda i: (i, 0, 0)),
            _full((128, 128)),
            _full(W1a.shape), _full(W1b.shape), _full(w1d.shape),
            _full(b1.shape), _full(W2.shape), _full(b2.shape),
            _full(pW1.shape), _full(pb1.shape), _full(pW2c.shape),
            _full(pb2.shape),
            _full(nW1a.shape), _full(nW1b.shape), _full(nb1.shape),
            _full(nW2.shape), _full(nb2.shape),
        ],
        out_specs=pl.BlockSpec((1, _P, 3), lambda i: (i, 0, 0)),
        out_shape=jax.ShapeDtypeStruct((_B, _P, 3), jnp.float32),
        scratch_shapes=[
            pltpu.VMEM((128, 128), jnp.float32),
            pltpu.VMEM((128, 3), jnp.float32),
            pltpu.VMEM((128, 128), jnp.float32),
            pltpu.VMEM((128, 1), jnp.float32),
            pltpu.VMEM((128, 128), jnp.float32),
            pltpu.VMEM((128, 1), jnp.float32),
            pltpu.VMEM((128, 3), jnp.float32),
            pltpu.VMEM((128, 128), jnp.float32),
            pltpu.VMEM((128, 128), jnp.float32),
            pltpu.VMEM((128, 128), jnp.float32),
        ],
        compiler_params=pltpu.CompilerParams(
            dimension_semantics=("arbitrary",)),
    )(zp, xp, sidc, sidr, qp1, emb_p, W1a, W1b, w1d, b1, W2, b2,
      pW1, pb1, pW2c, pb2, nW1a, nW1b, nb1, nW2, nb2)

    return out_p.reshape(_B * _P, 3)[inv]
